# baseline (speedup 1.0000x reference)
import functools

import jax
import jax.numpy as jnp
from jax import lax
from jax.experimental import pallas as pl
from jax.experimental.pallas import tpu as pltpu
from jax.experimental.pallas import tpu_sc as plsc

D = 64
B = 16384
NC = 2
NS = 16
L = 16
NW = NC * NS
BPW = B // NW

_mesh = plsc.VectorSubcoreMesh(core_axis_name="c", subcore_axis_name="s")


@functools.partial(
    pl.kernel,
    out_type=jax.ShapeDtypeStruct((B,), jnp.float32),
    mesh=_mesh,
    compiler_params=pltpu.CompilerParams(
        needs_layout_passes=False, use_tc_tiling_on_sc=False),
    scratch_types=[
        pltpu.VMEM((BPW,), jnp.int32),
        pltpu.VMEM((BPW,), jnp.int32),
        pltpu.VMEM((BPW, D), jnp.float32),
        pltpu.VMEM((BPW, D), jnp.float32),
        pltpu.VMEM((BPW * L,), jnp.float32),
        pltpu.VMEM((BPW,), jnp.float32),
        pltpu.SemaphoreType.DMA,
        pltpu.SemaphoreType.DMA,
    ],
)
def _sc_dots(target_hbm, context_hbm, w_in_hbm, w_out_hbm, out_hbm,
             idx_t, idx_c, rows_t, rows_c, parts, dots, sem_t, sem_c):
    wid = lax.axis_index("s") * NC + lax.axis_index("c")
    base = wid * BPW

    pltpu.sync_copy(target_hbm.at[pl.ds(base, BPW)], idx_t)
    pltpu.sync_copy(context_hbm.at[pl.ds(base, BPW)], idx_c)
    cp_t = pltpu.async_copy(w_in_hbm.at[idx_t], rows_t, sem_t)
    cp_c = pltpu.async_copy(w_out_hbm.at[idx_c], rows_c, sem_c)
    cp_t.wait()
    cp_c.wait()

    lanes = lax.iota(jnp.int32, L)

    def prod_body(r, carry):
        part = rows_t[r, pl.ds(0, L)] * rows_c[r, pl.ds(0, L)]
        for k in range(1, D // L):
            part = part + rows_t[r, pl.ds(k * L, L)] * rows_c[r, pl.ds(k * L, L)]
        parts[pl.ds(r * L, L)] = part
        return carry

    lax.fori_loop(0, BPW, prod_body, 0)

    def sum_body(g, carry):
        row_base = (g * L + lanes) * L
        acc = plsc.load_gather(parts, [row_base])
        for j in range(1, L):
            acc = acc + plsc.load_gather(parts, [row_base + j])
        dots[pl.ds(g * L, L)] = acc
        return carry

    lax.fori_loop(0, BPW // L, sum_body, 0)
    pltpu.sync_copy(dots, out_hbm.at[pl.ds(base, BPW)])


def _bce_body(z_ref, y_ref, out_ref):
    z = z_ref[...]
    y = y_ref[...].astype(jnp.float32)
    p = jax.nn.sigmoid(z)
    eps = 1e-12
    p = jnp.clip(p, eps, 1.0 - eps)
    loss = y * jnp.log(p) + (1.0 - y) * jnp.log(1.0 - p)
    out_ref[0, 0] = -jnp.sum(loss) / B


def kernel(target, context, labels, W_in, W_out):
    dots = _sc_dots(target, context, W_in, W_out)
    loss = pl.pallas_call(
        _bce_body,
        out_shape=jax.ShapeDtypeStruct((1, 1), jnp.float32),
        out_specs=pl.BlockSpec(memory_space=pltpu.SMEM),
    )(dots.reshape(128, 128), labels.reshape(128, 128))
    return loss[0, 0]
